# Initial kernel scaffold; baseline (speedup 1.0000x reference)
import jax
import jax.numpy as jnp
from jax.experimental import pallas as pl


def kernel(hidden_states, word_idxs, max_word_len):
    raise NotImplementedError("write your pallas kernel here")


# speedup vs baseline: 1.0048x; 1.0048x over previous
import jax
import jax.numpy as jnp
from jax import lax
from jax.experimental import pallas as pl
from jax.experimental.pallas import tpu as pltpu


def _ceil_to(x, m):
    return ((x + m - 1) // m) * m


def _pool_kernel(idx_ref, hs_ref, emb_ref):
    idx = idx_ref[0]
    starts = idx[:, 0:1]
    spans = idx[:, 1:2]
    Wp = idx.shape[0]

    hs = hs_ref[0]
    Sp = hs.shape[0]

    pos = lax.broadcasted_iota(jnp.int32, (Wp, Sp), 1)
    rel = pos - starts
    in_span = jnp.logical_and(rel >= 0, rel <= spans)
    sel = jnp.where(in_span, jnp.float32(1.0), jnp.float32(0.0)).astype(jnp.bfloat16)

    pooled = lax.dot_general(
        sel, hs.astype(jnp.bfloat16),
        (((1,), (0,)), ((), ())),
        preferred_element_type=jnp.float32)

    denom = jnp.maximum(spans + 1, 1).astype(jnp.float32)
    inv = jnp.where(spans >= 0, 1.0 / denom, 0.0)
    emb_ref[0] = pooled * inv


def kernel(hidden_states, word_idxs, max_word_len):
    B, S, H = hidden_states.shape
    W = word_idxs.shape[1]
    out_dtype = hidden_states.dtype

    word_iota = jnp.arange(W, dtype=jnp.int32)[None, :]
    valid = word_iota < max_word_len.astype(jnp.int32)[:, None]
    word_masks = valid.astype(jnp.int32)

    Hp = _ceil_to(H, 128)
    Wp = _ceil_to(W, 8)
    Sp = _ceil_to(S, 8)

    hs = hidden_states
    if (Sp, Hp) != (S, H):
        hs = jnp.pad(hs, ((0, 0), (0, Sp - S), (0, Hp - H)))

    starts = word_idxs[..., 0].astype(jnp.int32)
    spans = word_idxs[..., 1].astype(jnp.int32) - starts
    spans = jnp.where(valid, spans, -1)
    idx = jnp.stack([starts, spans], axis=-1)
    if Wp != W:
        idx = jnp.pad(idx, ((0, 0), (0, Wp - W), (0, 0)),
                      constant_values=-1)

    emb = pl.pallas_call(
        _pool_kernel,
        out_shape=jax.ShapeDtypeStruct((B, Wp, Hp), out_dtype),
        grid=(B,),
        in_specs=[
            pl.BlockSpec((1, Wp, 2), lambda b: (b, 0, 0)),
            pl.BlockSpec((1, Sp, Hp), lambda b: (b, 0, 0)),
        ],
        out_specs=pl.BlockSpec((1, Wp, Hp), lambda b: (b, 0, 0)),
        compiler_params=pltpu.CompilerParams(
            dimension_semantics=("parallel",),
            vmem_limit_bytes=48 * 1024 * 1024),
    )(idx, hs)

    if (Wp, Hp) != (W, H):
        emb = emb[:, :W, :H]
    return emb, word_masks


# speedup vs baseline: 1.5488x; 1.5414x over previous
import jax
import jax.numpy as jnp
from jax import lax
from jax.experimental import pallas as pl
from jax.experimental.pallas import tpu as pltpu


def _ceil_to(x, m):
    return ((x + m - 1) // m) * m


def _pool_kernel(idx_ref, hs_ref, emb_ref):
    idx = idx_ref[...]
    starts = idx[:, :, 0:1]
    spans = idx[:, :, 1:2]
    Bb, Wp, _ = idx.shape

    hs = hs_ref[...]
    Sp = hs.shape[1]

    pos = lax.broadcasted_iota(jnp.int32, (Bb, Wp, Sp), 2)
    rel = pos - starts
    in_span = jnp.logical_and(rel >= 0, rel <= spans)
    sel = jnp.where(in_span, jnp.float32(1.0), jnp.float32(0.0)).astype(jnp.bfloat16)

    pooled = lax.dot_general(
        sel, hs.astype(jnp.bfloat16),
        (((2,), (1,)), ((0,), (0,))),
        preferred_element_type=jnp.float32)

    denom = jnp.maximum(spans + 1, 1).astype(jnp.float32)
    inv = jnp.where(spans >= 0, 1.0 / denom, 0.0)
    emb_ref[...] = pooled * inv


def kernel(hidden_states, word_idxs, max_word_len):
    B, S, H = hidden_states.shape
    W = word_idxs.shape[1]
    out_dtype = hidden_states.dtype

    word_iota = jnp.arange(W, dtype=jnp.int32)[None, :]
    valid = word_iota < max_word_len.astype(jnp.int32)[:, None]
    word_masks = valid.astype(jnp.int32)

    Hp = _ceil_to(H, 128)
    Wp = _ceil_to(W, 8)
    Sp = _ceil_to(S, 8)

    hs = hidden_states
    if (Sp, Hp) != (S, H):
        hs = jnp.pad(hs, ((0, 0), (0, Sp - S), (0, Hp - H)))

    starts = word_idxs[..., 0].astype(jnp.int32)
    spans = word_idxs[..., 1].astype(jnp.int32) - starts
    spans = jnp.where(valid, spans, -1)
    idx = jnp.stack([starts, spans], axis=-1)
    if Wp != W:
        idx = jnp.pad(idx, ((0, 0), (0, Wp - W), (0, 0)),
                      constant_values=-1)

    Bb = 8
    while B % Bb != 0:
        Bb //= 2

    emb = pl.pallas_call(
        _pool_kernel,
        out_shape=jax.ShapeDtypeStruct((B, Wp, Hp), out_dtype),
        grid=(B // Bb,),
        in_specs=[
            pl.BlockSpec((Bb, Wp, 2), lambda b: (b, 0, 0)),
            pl.BlockSpec((Bb, Sp, Hp), lambda b: (b, 0, 0)),
        ],
        out_specs=pl.BlockSpec((Bb, Wp, Hp), lambda b: (b, 0, 0)),
        compiler_params=pltpu.CompilerParams(
            dimension_semantics=("parallel",),
            vmem_limit_bytes=56 * 1024 * 1024),
    )(idx, hs)

    if (Wp, Hp) != (W, H):
        emb = emb[:, :W, :H]
    return emb, word_masks


# speedup vs baseline: 1.5586x; 1.0063x over previous
import jax
import jax.numpy as jnp
from jax import lax
from jax.experimental import pallas as pl
from jax.experimental.pallas import tpu as pltpu


def _ceil_to(x, m):
    return ((x + m - 1) // m) * m


def _pool_kernel(idx_ref, hs_ref, emb_ref):
    idx = idx_ref[...]
    starts = idx[:, :, 0:1]
    spans = idx[:, :, 1:2]
    Bb, Wp, _ = idx.shape

    hs = hs_ref[...]
    Sp = hs.shape[1]

    pos = lax.broadcasted_iota(jnp.int32, (Bb, Wp, Sp), 2)
    rel = pos - starts
    in_span = jnp.logical_and(rel >= 0, rel <= spans)
    sel = jnp.where(in_span, jnp.float32(1.0), jnp.float32(0.0)).astype(jnp.bfloat16)

    pooled = lax.dot_general(
        sel, hs.astype(jnp.bfloat16),
        (((2,), (1,)), ((0,), (0,))),
        preferred_element_type=jnp.float32)

    denom = jnp.maximum(spans + 1, 1).astype(jnp.float32)
    inv = jnp.where(spans >= 0, 1.0 / denom, 0.0)
    emb_ref[...] = pooled * inv


def kernel(hidden_states, word_idxs, max_word_len):
    B, S, H = hidden_states.shape
    W = word_idxs.shape[1]
    out_dtype = hidden_states.dtype

    word_iota = jnp.arange(W, dtype=jnp.int32)[None, :]
    valid = word_iota < max_word_len.astype(jnp.int32)[:, None]
    word_masks = valid.astype(jnp.int32)

    Hp = _ceil_to(H, 128)
    Wp = _ceil_to(W, 8)
    Sp = _ceil_to(S, 8)

    hs = hidden_states
    if (Sp, Hp) != (S, H):
        hs = jnp.pad(hs, ((0, 0), (0, Sp - S), (0, Hp - H)))

    starts = word_idxs[..., 0].astype(jnp.int32)
    spans = word_idxs[..., 1].astype(jnp.int32) - starts
    spans = jnp.where(valid, spans, -1)
    idx = jnp.stack([starts, spans], axis=-1)
    if Wp != W:
        idx = jnp.pad(idx, ((0, 0), (0, Wp - W), (0, 0)),
                      constant_values=-1)

    Bb = 4
    while B % Bb != 0:
        Bb //= 2

    emb = pl.pallas_call(
        _pool_kernel,
        out_shape=jax.ShapeDtypeStruct((B, Wp, Hp), out_dtype),
        grid=(B // Bb,),
        in_specs=[
            pl.BlockSpec((Bb, Wp, 2), lambda b: (b, 0, 0)),
            pl.BlockSpec((Bb, Sp, Hp), lambda b: (b, 0, 0)),
        ],
        out_specs=pl.BlockSpec((Bb, Wp, Hp), lambda b: (b, 0, 0)),
        compiler_params=pltpu.CompilerParams(
            dimension_semantics=("parallel",),
            vmem_limit_bytes=56 * 1024 * 1024),
    )(idx, hs)

    if (Wp, Hp) != (W, H):
        emb = emb[:, :W, :H]
    return emb, word_masks
